# baseline (speedup 1.0000x reference)
import dataclasses
import functools

import jax
import jax.numpy as jnp
from jax import lax
from jax.experimental import pallas as pl
from jax.experimental.pallas import tpu as pltpu
from jax.experimental.pallas import tpu_sc as plsc

N = 10000
D = 128
N_PAD = 10240
E = 320000
NC, NS = 2, 16
NW = NC * NS
CHUNK = 128
EDGES_PER_TILE = 10240
CHUNKS_PER_TILE = EDGES_PER_TILE // CHUNK
C0_CHUNKS = 112
C1_CHUNKS = 2 * CHUNKS_PER_TILE - C0_CHUNKS
NBUF = 2
E_PAD = NW * EDGES_PER_TILE
N_CHUNKS = E_PAD // CHUNK
ROWS_PER_SUBCORE = N_PAD // NS
DEG_W = 1

_mesh = plsc.VectorSubcoreMesh(core_axis_name="c", subcore_axis_name="s")

_sc_params = pltpu.CompilerParams()
if "needs_layout_passes" in pltpu.CompilerParams.__dataclass_fields__:
    _sc_params = dataclasses.replace(_sc_params, needs_layout_passes=False)


@functools.partial(
    pl.kernel,
    mesh=_mesh,
    compiler_params=_sc_params,
    out_type=jax.ShapeDtypeStruct((NC * N_PAD,), jnp.float32),
    scratch_types=[
        pltpu.VMEM((CHUNK,), jnp.int32),
        pltpu.VMEM((N_PAD,), jnp.float32),
        pltpu.VMEM((NS * ROWS_PER_SUBCORE,), jnp.float32),
        pltpu.VMEM((ROWS_PER_SUBCORE,), jnp.float32),
        pltpu.VMEM_SHARED((NS * N_PAD,), jnp.float32),
    ],
)
def _deg_kernel(dst_hbm, deg_hbm, dst_v, cnt_v, gath_v, res_v, stage_sh):
    c = lax.axis_index("c")
    s = lax.axis_index("s")
    base = (c * NS + s) * EDGES_PER_TILE
    r0 = s * ROWS_PER_SUBCORE

    @pl.loop(0, N_PAD, step=16)
    def _(k):
        cnt_v.at[pl.ds(k, 16)][...] = jnp.zeros((16,), jnp.float32)

    ones = jnp.ones((16,), jnp.float32)

    @pl.loop(0, CHUNKS_PER_TILE)
    def _(j):
        pltpu.sync_copy(dst_hbm.at[pl.ds(base + j * CHUNK, CHUNK)], dst_v)

        @pl.loop(0, CHUNK, step=16)
        def _(k):
            plsc.addupdate_scatter(cnt_v, [dst_v[pl.ds(k, 16)]], ones)

    pltpu.sync_copy(cnt_v, stage_sh.at[pl.ds(s * N_PAD, N_PAD)])
    plsc.subcore_barrier()
    for r in range(NS):
        pltpu.sync_copy(
            stage_sh.at[pl.ds(r * N_PAD + r0, ROWS_PER_SUBCORE)],
            gath_v.at[pl.ds(r * ROWS_PER_SUBCORE, ROWS_PER_SUBCORE)])

    @pl.loop(0, ROWS_PER_SUBCORE, step=16)
    def _(k):
        acc = jnp.zeros((16,), jnp.float32)
        for r in range(NS):
            acc = acc + gath_v[pl.ds(r * ROWS_PER_SUBCORE + k, 16)]
        res_v.at[pl.ds(k, 16)][...] = acc

    pltpu.sync_copy(res_v, deg_hbm.at[pl.ds(c * N_PAD + r0, ROWS_PER_SUBCORE)])


@functools.partial(
    pl.kernel,
    mesh=_mesh,
    out_type=jax.ShapeDtypeStruct((NC * N_PAD, D), jnp.float32),
    scratch_types=(
        [pltpu.VMEM((CHUNK,), jnp.int32) for _ in range(2 * NBUF)]
        + [pltpu.VMEM((CHUNK, D), jnp.float32) for _ in range(NBUF)]
        + [pltpu.VMEM_SHARED((N_PAD, D), jnp.float32)]
        + [pltpu.SemaphoreType.DMA for _ in range(2 * NBUF)]
    ),
)
def _scatter_kernel(hs_hbm, src_hbm, dst_hbm, zero_hbm, out_hbm, *scratch):
    src_v = scratch[0:NBUF]
    dst_v = scratch[NBUF:2 * NBUF]
    rows_v = scratch[2 * NBUF:3 * NBUF]
    acc_sh = scratch[3 * NBUF]
    semg = scratch[3 * NBUF + 1:3 * NBUF + 1 + NBUF]
    semi = scratch[3 * NBUF + 1 + NBUF:3 * NBUF + 1 + 2 * NBUF]

    c = lax.axis_index("c")
    s = lax.axis_index("s")
    nch = jnp.where(c == 0, C0_CHUNKS, C1_CHUNKS)
    base = jnp.where(
        c == 0, s * (C0_CHUNKS * CHUNK),
        NS * (C0_CHUNKS * CHUNK) + s * (C1_CHUNKS * CHUNK))

    r0 = s * ROWS_PER_SUBCORE
    pltpu.sync_copy(zero_hbm, acc_sh.at[pl.ds(r0, ROWS_PER_SUBCORE)])
    plsc.subcore_barrier()

    def run(hs_hbm):
        for k in range(NBUF - 1):
            pltpu.sync_copy(src_hbm.at[pl.ds(base + k * CHUNK, CHUNK)], src_v[k])
            pltpu.sync_copy(dst_hbm.at[pl.ds(base + k * CHUNK, CHUNK)], dst_v[k])
            pltpu.async_copy(hs_hbm.at[src_v[k]], rows_v[k], semg[k])
        kb = NBUF - 1
        pltpu.async_copy(
            src_hbm.at[pl.ds(base + kb * CHUNK, CHUNK)], src_v[kb], semi[kb])
        pltpu.async_copy(
            dst_hbm.at[pl.ds(base + kb * CHUNK, CHUNK)], dst_v[kb], semi[kb])

        @pl.loop(0, nch, step=NBUF)
        def _(g):
            for b in range(NBUF):
                j = g + b
                o = (b + NBUF - 1) % NBUF

                @pl.when(j + NBUF - 1 < nch)
                def _():
                    off = base + (j + NBUF - 1) * CHUNK
                    pltpu.make_async_copy(
                        src_hbm.at[pl.ds(off, CHUNK)], src_v[o], semi[o]).wait()
                    pltpu.make_async_copy(
                        dst_hbm.at[pl.ds(off, CHUNK)], dst_v[o], semi[o]).wait()
                    pltpu.async_copy(hs_hbm.at[src_v[o]], rows_v[o], semg[o])

                pltpu.make_async_copy(
                    hs_hbm.at[src_v[b]], rows_v[b], semg[b]).wait()
                pltpu.sync_copy(rows_v[b], acc_sh.at[dst_v[b]], add=True)

                @pl.when(j + NBUF < nch)
                def _():
                    off = base + (j + NBUF) * CHUNK
                    pltpu.async_copy(
                        src_hbm.at[pl.ds(off, CHUNK)], src_v[b], semi[b])
                    pltpu.async_copy(
                        dst_hbm.at[pl.ds(off, CHUNK)], dst_v[b], semi[b])

    run(hs_hbm)

    plsc.subcore_barrier()
    pltpu.sync_copy(acc_sh.at[pl.ds(r0, ROWS_PER_SUBCORE)],
                    out_hbm.at[pl.ds(c * N_PAD + r0, ROWS_PER_SUBCORE)])


_BLK = 1280
_GRID = N_PAD // _BLK


def _row_spec(w=D):
    return pl.BlockSpec((_BLK, w), lambda i: (i, 0))


def _fixed_spec(shape):
    return pl.BlockSpec(shape, lambda i: tuple(0 for _ in shape))


def _dinv(dga_ref, dgb_ref):
    deg = dga_ref[...] + dgb_ref[...] + 1.0
    return lax.rsqrt(deg)


def _mm_body(x_ref, w_ref, o_ref):
    o_ref[...] = lax.dot_general(
        x_ref[...], w_ref[...], (((1,), (0,)), ((), ())),
        precision=lax.Precision.HIGHEST, preferred_element_type=jnp.float32)


_mm = pl.pallas_call(
    _mm_body,
    grid=(_GRID,),
    in_specs=[_row_spec(), _fixed_spec((D, D))],
    out_specs=_row_spec(),
    out_shape=jax.ShapeDtypeStruct((N_PAD, D), jnp.float32),
)


def _scale_body(h_ref, dga_ref, dgb_ref, o_ref):
    o_ref[...] = _dinv(dga_ref, dgb_ref) * h_ref[...]


_scale = pl.pallas_call(
    _scale_body,
    grid=(_GRID,),
    in_specs=[_row_spec(), _row_spec(DEG_W), _row_spec(DEG_W)],
    out_specs=_row_spec(),
    out_shape=jax.ShapeDtypeStruct((N_PAD, D), jnp.float32),
)


def _mid_body(sa_ref, sb_ref, hs_ref, dga_ref, dgb_ref, b_ref, w_ref, o_ref):
    dinv = _dinv(dga_ref, dgb_ref)
    t = jnp.maximum(dinv * (sa_ref[...] + sb_ref[...] + hs_ref[...]) + b_ref[...], 0.0)
    o_ref[...] = dinv * lax.dot_general(
        t, w_ref[...], (((1,), (0,)), ((), ())),
        precision=lax.Precision.HIGHEST, preferred_element_type=jnp.float32)


_mid = pl.pallas_call(
    _mid_body,
    grid=(_GRID,),
    in_specs=[_row_spec(), _row_spec(), _row_spec(), _row_spec(DEG_W),
              _row_spec(DEG_W), _fixed_spec((1, D)), _fixed_spec((D, D))],
    out_specs=_row_spec(),
    out_shape=jax.ShapeDtypeStruct((N_PAD, D), jnp.float32),
)


def _out_body(sa_ref, sb_ref, hs_ref, dga_ref, dgb_ref, b_ref, o_ref):
    dinv = _dinv(dga_ref, dgb_ref)
    o_ref[...] = dinv * (sa_ref[...] + sb_ref[...] + hs_ref[...]) + b_ref[...]


_out = pl.pallas_call(
    _out_body,
    grid=(_GRID,),
    in_specs=[_row_spec(), _row_spec(), _row_spec(), _row_spec(DEG_W),
              _row_spec(DEG_W), _fixed_spec((1, D))],
    out_specs=_row_spec(),
    out_shape=jax.ShapeDtypeStruct((N_PAD, D), jnp.float32),
)


def kernel(x, edge_index, W1, b1, W2, b2):
    src = edge_index[0].astype(jnp.int32)
    dst = edge_index[1].astype(jnp.int32)
    pad = jnp.full((E_PAD - E,), N, jnp.int32)
    src_p = jnp.concatenate([src, pad])
    dst_p = jnp.concatenate([dst, pad])
    x_p = jnp.pad(x, ((0, N_PAD - N), (0, 0)))
    zeroD = jnp.zeros((ROWS_PER_SUBCORE, D), jnp.float32)

    deg2 = _deg_kernel(dst_p).reshape(NC, N_PAD, DEG_W)
    dga, dgb = deg2[0], deg2[1]

    h1 = _mm(x_p, W1)
    hs1 = _scale(h1, dga, dgb)
    s1 = _scatter_kernel(hs1, src_p, dst_p, zeroD).reshape(NC, N_PAD, D)
    hs2 = _mid(s1[0], s1[1], hs1, dga, dgb, b1.reshape(1, D), W2)
    s2 = _scatter_kernel(hs2, src_p, dst_p, zeroD).reshape(NC, N_PAD, D)
    out = _out(s2[0], s2[1], hs2, dga, dgb, b2.reshape(1, D))
    return out[:N]


# speedup vs baseline: 1.0089x; 1.0089x over previous
import dataclasses
import functools

import jax
import jax.numpy as jnp
from jax import lax
from jax.experimental import pallas as pl
from jax.experimental.pallas import tpu as pltpu
from jax.experimental.pallas import tpu_sc as plsc

N = 10000
D = 128
N_PAD = 10240
E = 320000
NC, NS = 2, 16
NW = NC * NS
CHUNK = 128
EDGES_PER_TILE = 10240
CHUNKS_PER_TILE = EDGES_PER_TILE // CHUNK
C0_CHUNKS = 128
C1_CHUNKS = 2 * CHUNKS_PER_TILE - C0_CHUNKS
NBUF = 2
E_PAD = NW * EDGES_PER_TILE
N_CHUNKS = E_PAD // CHUNK
ROWS_PER_SUBCORE = N_PAD // NS
DEG_W = 1

_mesh = plsc.VectorSubcoreMesh(core_axis_name="c", subcore_axis_name="s")

_sc_params = pltpu.CompilerParams()
if "needs_layout_passes" in pltpu.CompilerParams.__dataclass_fields__:
    _sc_params = dataclasses.replace(_sc_params, needs_layout_passes=False)


@functools.partial(
    pl.kernel,
    mesh=_mesh,
    compiler_params=_sc_params,
    out_type=jax.ShapeDtypeStruct((NC * N_PAD,), jnp.float32),
    scratch_types=[
        pltpu.VMEM((CHUNK,), jnp.int32),
        pltpu.VMEM((N_PAD,), jnp.float32),
        pltpu.VMEM((NS * ROWS_PER_SUBCORE,), jnp.float32),
        pltpu.VMEM((ROWS_PER_SUBCORE,), jnp.float32),
        pltpu.VMEM_SHARED((NS * N_PAD,), jnp.float32),
    ],
)
def _deg_kernel(dst_hbm, deg_hbm, dst_v, cnt_v, gath_v, res_v, stage_sh):
    c = lax.axis_index("c")
    s = lax.axis_index("s")
    base = (c * NS + s) * EDGES_PER_TILE
    r0 = s * ROWS_PER_SUBCORE

    @pl.loop(0, N_PAD, step=16)
    def _(k):
        cnt_v.at[pl.ds(k, 16)][...] = jnp.zeros((16,), jnp.float32)

    ones = jnp.ones((16,), jnp.float32)

    @pl.loop(0, CHUNKS_PER_TILE)
    def _(j):
        pltpu.sync_copy(dst_hbm.at[pl.ds(base + j * CHUNK, CHUNK)], dst_v)

        @pl.loop(0, CHUNK, step=16)
        def _(k):
            plsc.addupdate_scatter(cnt_v, [dst_v[pl.ds(k, 16)]], ones)

    pltpu.sync_copy(cnt_v, stage_sh.at[pl.ds(s * N_PAD, N_PAD)])
    plsc.subcore_barrier()
    for r in range(NS):
        pltpu.sync_copy(
            stage_sh.at[pl.ds(r * N_PAD + r0, ROWS_PER_SUBCORE)],
            gath_v.at[pl.ds(r * ROWS_PER_SUBCORE, ROWS_PER_SUBCORE)])

    @pl.loop(0, ROWS_PER_SUBCORE, step=16)
    def _(k):
        acc = jnp.zeros((16,), jnp.float32)
        for r in range(NS):
            acc = acc + gath_v[pl.ds(r * ROWS_PER_SUBCORE + k, 16)]
        res_v.at[pl.ds(k, 16)][...] = acc

    pltpu.sync_copy(res_v, deg_hbm.at[pl.ds(c * N_PAD + r0, ROWS_PER_SUBCORE)])


@functools.partial(
    pl.kernel,
    mesh=_mesh,
    out_type=jax.ShapeDtypeStruct((NC * N_PAD, D), jnp.float32),
    scratch_types=(
        [pltpu.VMEM((CHUNK,), jnp.int32) for _ in range(2 * NBUF)]
        + [pltpu.VMEM((CHUNK, D), jnp.float32) for _ in range(NBUF)]
        + [pltpu.VMEM_SHARED((N_PAD, D), jnp.float32)]
        + [pltpu.SemaphoreType.DMA for _ in range(2 * NBUF)]
    ),
)
def _scatter_kernel(hs_hbm, src_hbm, dst_hbm, zero_hbm, out_hbm, *scratch):
    src_v = scratch[0:NBUF]
    dst_v = scratch[NBUF:2 * NBUF]
    rows_v = scratch[2 * NBUF:3 * NBUF]
    acc_sh = scratch[3 * NBUF]
    semg = scratch[3 * NBUF + 1:3 * NBUF + 1 + NBUF]
    semi = scratch[3 * NBUF + 1 + NBUF:3 * NBUF + 1 + 2 * NBUF]

    c = lax.axis_index("c")
    s = lax.axis_index("s")
    nch = jnp.where(c == 0, C0_CHUNKS, C1_CHUNKS)
    base = jnp.where(
        c == 0, s * (C0_CHUNKS * CHUNK),
        NS * (C0_CHUNKS * CHUNK) + s * (C1_CHUNKS * CHUNK))

    r0 = s * ROWS_PER_SUBCORE
    pltpu.sync_copy(zero_hbm, acc_sh.at[pl.ds(r0, ROWS_PER_SUBCORE)])
    plsc.subcore_barrier()

    def run(hs_hbm):
        for k in range(NBUF - 1):
            pltpu.sync_copy(src_hbm.at[pl.ds(base + k * CHUNK, CHUNK)], src_v[k])
            pltpu.sync_copy(dst_hbm.at[pl.ds(base + k * CHUNK, CHUNK)], dst_v[k])
            pltpu.async_copy(hs_hbm.at[src_v[k]], rows_v[k], semg[k])
        kb = NBUF - 1
        pltpu.async_copy(
            src_hbm.at[pl.ds(base + kb * CHUNK, CHUNK)], src_v[kb], semi[kb])
        pltpu.async_copy(
            dst_hbm.at[pl.ds(base + kb * CHUNK, CHUNK)], dst_v[kb], semi[kb])

        @pl.loop(0, nch, step=NBUF)
        def _(g):
            for b in range(NBUF):
                j = g + b
                o = (b + NBUF - 1) % NBUF

                @pl.when(j + NBUF - 1 < nch)
                def _():
                    off = base + (j + NBUF - 1) * CHUNK
                    pltpu.make_async_copy(
                        src_hbm.at[pl.ds(off, CHUNK)], src_v[o], semi[o]).wait()
                    pltpu.make_async_copy(
                        dst_hbm.at[pl.ds(off, CHUNK)], dst_v[o], semi[o]).wait()
                    pltpu.async_copy(hs_hbm.at[src_v[o]], rows_v[o], semg[o])

                pltpu.make_async_copy(
                    hs_hbm.at[src_v[b]], rows_v[b], semg[b]).wait()
                pltpu.sync_copy(rows_v[b], acc_sh.at[dst_v[b]], add=True)

                @pl.when(j + NBUF < nch)
                def _():
                    off = base + (j + NBUF) * CHUNK
                    pltpu.async_copy(
                        src_hbm.at[pl.ds(off, CHUNK)], src_v[b], semi[b])
                    pltpu.async_copy(
                        dst_hbm.at[pl.ds(off, CHUNK)], dst_v[b], semi[b])

    run(hs_hbm)

    plsc.subcore_barrier()
    pltpu.sync_copy(acc_sh.at[pl.ds(r0, ROWS_PER_SUBCORE)],
                    out_hbm.at[pl.ds(c * N_PAD + r0, ROWS_PER_SUBCORE)])


_BLK = 1280
_GRID = N_PAD // _BLK


def _row_spec(w=D):
    return pl.BlockSpec((_BLK, w), lambda i: (i, 0))


def _fixed_spec(shape):
    return pl.BlockSpec(shape, lambda i: tuple(0 for _ in shape))


def _dinv(dga_ref, dgb_ref):
    deg = dga_ref[...] + dgb_ref[...] + 1.0
    return lax.rsqrt(deg)


def _mm_body(x_ref, w_ref, o_ref):
    o_ref[...] = lax.dot_general(
        x_ref[...], w_ref[...], (((1,), (0,)), ((), ())),
        precision=lax.Precision.HIGHEST, preferred_element_type=jnp.float32)


_mm = pl.pallas_call(
    _mm_body,
    grid=(_GRID,),
    in_specs=[_row_spec(), _fixed_spec((D, D))],
    out_specs=_row_spec(),
    out_shape=jax.ShapeDtypeStruct((N_PAD, D), jnp.float32),
)


def _scale_body(h_ref, dga_ref, dgb_ref, o_ref):
    o_ref[...] = _dinv(dga_ref, dgb_ref) * h_ref[...]


_scale = pl.pallas_call(
    _scale_body,
    grid=(_GRID,),
    in_specs=[_row_spec(), _row_spec(DEG_W), _row_spec(DEG_W)],
    out_specs=_row_spec(),
    out_shape=jax.ShapeDtypeStruct((N_PAD, D), jnp.float32),
)


def _mid_body(sa_ref, sb_ref, hs_ref, dga_ref, dgb_ref, b_ref, w_ref, o_ref):
    dinv = _dinv(dga_ref, dgb_ref)
    t = jnp.maximum(dinv * (sa_ref[...] + sb_ref[...] + hs_ref[...]) + b_ref[...], 0.0)
    o_ref[...] = dinv * lax.dot_general(
        t, w_ref[...], (((1,), (0,)), ((), ())),
        precision=lax.Precision.HIGHEST, preferred_element_type=jnp.float32)


_mid = pl.pallas_call(
    _mid_body,
    grid=(_GRID,),
    in_specs=[_row_spec(), _row_spec(), _row_spec(), _row_spec(DEG_W),
              _row_spec(DEG_W), _fixed_spec((1, D)), _fixed_spec((D, D))],
    out_specs=_row_spec(),
    out_shape=jax.ShapeDtypeStruct((N_PAD, D), jnp.float32),
)


def _out_body(sa_ref, sb_ref, hs_ref, dga_ref, dgb_ref, b_ref, o_ref):
    dinv = _dinv(dga_ref, dgb_ref)
    o_ref[...] = dinv * (sa_ref[...] + sb_ref[...] + hs_ref[...]) + b_ref[...]


_out = pl.pallas_call(
    _out_body,
    grid=(_GRID,),
    in_specs=[_row_spec(), _row_spec(), _row_spec(), _row_spec(DEG_W),
              _row_spec(DEG_W), _fixed_spec((1, D))],
    out_specs=_row_spec(),
    out_shape=jax.ShapeDtypeStruct((N_PAD, D), jnp.float32),
)


def kernel(x, edge_index, W1, b1, W2, b2):
    src = edge_index[0].astype(jnp.int32)
    dst = edge_index[1].astype(jnp.int32)
    pad = jnp.full((E_PAD - E,), N, jnp.int32)
    src_p = jnp.concatenate([src, pad])
    dst_p = jnp.concatenate([dst, pad])
    x_p = jnp.pad(x, ((0, N_PAD - N), (0, 0)))
    zeroD = jnp.zeros((ROWS_PER_SUBCORE, D), jnp.float32)

    deg2 = _deg_kernel(dst_p).reshape(NC, N_PAD, DEG_W)
    dga, dgb = deg2[0], deg2[1]

    h1 = _mm(x_p, W1)
    hs1 = _scale(h1, dga, dgb)
    s1 = _scatter_kernel(hs1, src_p, dst_p, zeroD).reshape(NC, N_PAD, D)
    hs2 = _mid(s1[0], s1[1], hs1, dga, dgb, b1.reshape(1, D), W2)
    s2 = _scatter_kernel(hs2, src_p, dst_p, zeroD).reshape(NC, N_PAD, D)
    out = _out(s2[0], s2[1], hs2, dga, dgb, b2.reshape(1, D))
    return out[:N]


# speedup vs baseline: 1.0570x; 1.0476x over previous
import dataclasses
import functools

import jax
import jax.numpy as jnp
from jax import lax
from jax.experimental import pallas as pl
from jax.experimental.pallas import tpu as pltpu
from jax.experimental.pallas import tpu_sc as plsc

N = 10000
D = 128
N_PAD = 10240
E = 320000
NC, NS = 2, 16
NW = NC * NS
CHUNK = 128
EDGES_PER_TILE = 10240
CHUNKS_PER_TILE = EDGES_PER_TILE // CHUNK
C0_CHUNKS = 136
C1_CHUNKS = 2 * CHUNKS_PER_TILE - C0_CHUNKS
NBUF = 2
E_PAD = NW * EDGES_PER_TILE
N_CHUNKS = E_PAD // CHUNK
ROWS_PER_SUBCORE = N_PAD // NS
DEG_W = 1

_mesh = plsc.VectorSubcoreMesh(core_axis_name="c", subcore_axis_name="s")

_sc_params = pltpu.CompilerParams()
if "needs_layout_passes" in pltpu.CompilerParams.__dataclass_fields__:
    _sc_params = dataclasses.replace(_sc_params, needs_layout_passes=False)


@functools.partial(
    pl.kernel,
    mesh=_mesh,
    compiler_params=_sc_params,
    out_type=jax.ShapeDtypeStruct((NC * N_PAD,), jnp.float32),
    scratch_types=[
        pltpu.VMEM((CHUNK,), jnp.int32),
        pltpu.VMEM((N_PAD,), jnp.float32),
        pltpu.VMEM((NS * ROWS_PER_SUBCORE,), jnp.float32),
        pltpu.VMEM((ROWS_PER_SUBCORE,), jnp.float32),
        pltpu.VMEM_SHARED((NS * N_PAD,), jnp.float32),
    ],
)
def _deg_kernel(dst_hbm, deg_hbm, dst_v, cnt_v, gath_v, res_v, stage_sh):
    c = lax.axis_index("c")
    s = lax.axis_index("s")
    base = (c * NS + s) * EDGES_PER_TILE
    r0 = s * ROWS_PER_SUBCORE

    @pl.loop(0, N_PAD, step=16)
    def _(k):
        cnt_v.at[pl.ds(k, 16)][...] = jnp.zeros((16,), jnp.float32)

    ones = jnp.ones((16,), jnp.float32)

    @pl.loop(0, CHUNKS_PER_TILE)
    def _(j):
        pltpu.sync_copy(dst_hbm.at[pl.ds(base + j * CHUNK, CHUNK)], dst_v)

        @pl.loop(0, CHUNK, step=16)
        def _(k):
            plsc.addupdate_scatter(cnt_v, [dst_v[pl.ds(k, 16)]], ones)

    pltpu.sync_copy(cnt_v, stage_sh.at[pl.ds(s * N_PAD, N_PAD)])
    plsc.subcore_barrier()
    for r in range(NS):
        pltpu.sync_copy(
            stage_sh.at[pl.ds(r * N_PAD + r0, ROWS_PER_SUBCORE)],
            gath_v.at[pl.ds(r * ROWS_PER_SUBCORE, ROWS_PER_SUBCORE)])

    @pl.loop(0, ROWS_PER_SUBCORE, step=16)
    def _(k):
        acc = jnp.zeros((16,), jnp.float32)
        for r in range(NS):
            acc = acc + gath_v[pl.ds(r * ROWS_PER_SUBCORE + k, 16)]
        res_v.at[pl.ds(k, 16)][...] = acc

    pltpu.sync_copy(res_v, deg_hbm.at[pl.ds(c * N_PAD + r0, ROWS_PER_SUBCORE)])


@functools.partial(
    pl.kernel,
    mesh=_mesh,
    out_type=jax.ShapeDtypeStruct((NC * N_PAD, D), jnp.float32),
    scratch_types=(
        [pltpu.VMEM((CHUNK,), jnp.int32) for _ in range(2 * NBUF)]
        + [pltpu.VMEM((CHUNK, D), jnp.float32) for _ in range(NBUF)]
        + [pltpu.VMEM_SHARED((N_PAD, D), jnp.float32)]
        + [pltpu.SemaphoreType.DMA for _ in range(2 * NBUF)]
    ),
)
def _scatter_kernel(hs_hbm, src_hbm, dst_hbm, zero_hbm, out_hbm, *scratch):
    src_v = scratch[0:NBUF]
    dst_v = scratch[NBUF:2 * NBUF]
    rows_v = scratch[2 * NBUF:3 * NBUF]
    acc_sh = scratch[3 * NBUF]
    semg = scratch[3 * NBUF + 1:3 * NBUF + 1 + NBUF]
    semi = scratch[3 * NBUF + 1 + NBUF:3 * NBUF + 1 + 2 * NBUF]

    c = lax.axis_index("c")
    s = lax.axis_index("s")
    nch = jnp.where(c == 0, C0_CHUNKS, C1_CHUNKS)
    base = jnp.where(
        c == 0, s * (C0_CHUNKS * CHUNK),
        NS * (C0_CHUNKS * CHUNK) + s * (C1_CHUNKS * CHUNK))

    r0 = s * ROWS_PER_SUBCORE
    pltpu.sync_copy(zero_hbm, acc_sh.at[pl.ds(r0, ROWS_PER_SUBCORE)])
    plsc.subcore_barrier()

    def run(hs_hbm):
        for k in range(NBUF - 1):
            pltpu.sync_copy(src_hbm.at[pl.ds(base + k * CHUNK, CHUNK)], src_v[k])
            pltpu.sync_copy(dst_hbm.at[pl.ds(base + k * CHUNK, CHUNK)], dst_v[k])
            pltpu.async_copy(hs_hbm.at[src_v[k]], rows_v[k], semg[k])
        kb = NBUF - 1
        pltpu.async_copy(
            src_hbm.at[pl.ds(base + kb * CHUNK, CHUNK)], src_v[kb], semi[kb])
        pltpu.async_copy(
            dst_hbm.at[pl.ds(base + kb * CHUNK, CHUNK)], dst_v[kb], semi[kb])

        @pl.loop(0, nch, step=NBUF)
        def _(g):
            for b in range(NBUF):
                j = g + b
                o = (b + NBUF - 1) % NBUF

                @pl.when(j + NBUF - 1 < nch)
                def _():
                    off = base + (j + NBUF - 1) * CHUNK
                    pltpu.make_async_copy(
                        src_hbm.at[pl.ds(off, CHUNK)], src_v[o], semi[o]).wait()
                    pltpu.make_async_copy(
                        dst_hbm.at[pl.ds(off, CHUNK)], dst_v[o], semi[o]).wait()
                    pltpu.async_copy(hs_hbm.at[src_v[o]], rows_v[o], semg[o])

                pltpu.make_async_copy(
                    hs_hbm.at[src_v[b]], rows_v[b], semg[b]).wait()
                pltpu.sync_copy(rows_v[b], acc_sh.at[dst_v[b]], add=True)

                @pl.when(j + NBUF < nch)
                def _():
                    off = base + (j + NBUF) * CHUNK
                    pltpu.async_copy(
                        src_hbm.at[pl.ds(off, CHUNK)], src_v[b], semi[b])
                    pltpu.async_copy(
                        dst_hbm.at[pl.ds(off, CHUNK)], dst_v[b], semi[b])

    run(hs_hbm)

    plsc.subcore_barrier()
    pltpu.sync_copy(acc_sh.at[pl.ds(r0, ROWS_PER_SUBCORE)],
                    out_hbm.at[pl.ds(c * N_PAD + r0, ROWS_PER_SUBCORE)])


_BLK = 1280
_GRID = N_PAD // _BLK


def _row_spec(w=D):
    return pl.BlockSpec((_BLK, w), lambda i: (i, 0))


def _fixed_spec(shape):
    return pl.BlockSpec(shape, lambda i: tuple(0 for _ in shape))


def _dinv(dga_ref, dgb_ref):
    deg = dga_ref[...] + dgb_ref[...] + 1.0
    return lax.rsqrt(deg)


def _mm_body(x_ref, w_ref, o_ref):
    o_ref[...] = lax.dot_general(
        x_ref[...], w_ref[...], (((1,), (0,)), ((), ())),
        precision=lax.Precision.HIGHEST, preferred_element_type=jnp.float32)


_mm = pl.pallas_call(
    _mm_body,
    grid=(_GRID,),
    in_specs=[_row_spec(), _fixed_spec((D, D))],
    out_specs=_row_spec(),
    out_shape=jax.ShapeDtypeStruct((N_PAD, D), jnp.float32),
)


def _scale_body(h_ref, dga_ref, dgb_ref, o_ref):
    o_ref[...] = _dinv(dga_ref, dgb_ref) * h_ref[...]


_scale = pl.pallas_call(
    _scale_body,
    grid=(_GRID,),
    in_specs=[_row_spec(), _row_spec(DEG_W), _row_spec(DEG_W)],
    out_specs=_row_spec(),
    out_shape=jax.ShapeDtypeStruct((N_PAD, D), jnp.float32),
)


def _mid_body(sa_ref, sb_ref, hs_ref, dga_ref, dgb_ref, b_ref, w_ref, o_ref):
    dinv = _dinv(dga_ref, dgb_ref)
    t = jnp.maximum(dinv * (sa_ref[...] + sb_ref[...] + hs_ref[...]) + b_ref[...], 0.0)
    o_ref[...] = dinv * lax.dot_general(
        t, w_ref[...], (((1,), (0,)), ((), ())),
        precision=lax.Precision.HIGHEST, preferred_element_type=jnp.float32)


_mid = pl.pallas_call(
    _mid_body,
    grid=(_GRID,),
    in_specs=[_row_spec(), _row_spec(), _row_spec(), _row_spec(DEG_W),
              _row_spec(DEG_W), _fixed_spec((1, D)), _fixed_spec((D, D))],
    out_specs=_row_spec(),
    out_shape=jax.ShapeDtypeStruct((N_PAD, D), jnp.float32),
)


def _out_body(sa_ref, sb_ref, hs_ref, dga_ref, dgb_ref, b_ref, o_ref):
    dinv = _dinv(dga_ref, dgb_ref)
    o_ref[...] = dinv * (sa_ref[...] + sb_ref[...] + hs_ref[...]) + b_ref[...]


_out = pl.pallas_call(
    _out_body,
    grid=(_GRID,),
    in_specs=[_row_spec(), _row_spec(), _row_spec(), _row_spec(DEG_W),
              _row_spec(DEG_W), _fixed_spec((1, D))],
    out_specs=_row_spec(),
    out_shape=jax.ShapeDtypeStruct((N_PAD, D), jnp.float32),
)


def kernel(x, edge_index, W1, b1, W2, b2):
    src = edge_index[0].astype(jnp.int32)
    dst = edge_index[1].astype(jnp.int32)
    pad = jnp.full((E_PAD - E,), N, jnp.int32)
    src_p = jnp.concatenate([src, pad])
    dst_p = jnp.concatenate([dst, pad])
    x_p = jnp.pad(x, ((0, N_PAD - N), (0, 0)))
    zeroD = jnp.zeros((ROWS_PER_SUBCORE, D), jnp.float32)

    deg2 = _deg_kernel(dst_p).reshape(NC, N_PAD, DEG_W)
    dga, dgb = deg2[0], deg2[1]

    h1 = _mm(x_p, W1)
    hs1 = _scale(h1, dga, dgb)
    s1 = _scatter_kernel(hs1, src_p, dst_p, zeroD).reshape(NC, N_PAD, D)
    hs2 = _mid(s1[0], s1[1], hs1, dga, dgb, b1.reshape(1, D), W2)
    s2 = _scatter_kernel(hs2, src_p, dst_p, zeroD).reshape(NC, N_PAD, D)
    out = _out(s2[0], s2[1], hs2, dga, dgb, b2.reshape(1, D))
    return out[:N]


# speedup vs baseline: 1.1871x; 1.1231x over previous
import dataclasses
import functools

import jax
import jax.numpy as jnp
from jax import lax
from jax.experimental import pallas as pl
from jax.experimental.pallas import tpu as pltpu
from jax.experimental.pallas import tpu_sc as plsc

N = 10000
D = 128
N_PAD = 10240
E = 320000
NC, NS = 2, 16
NW = NC * NS
CHUNK = 128
EDGES_PER_TILE = 10240
CHUNKS_PER_TILE = EDGES_PER_TILE // CHUNK
C0_CHUNKS = 144
C1_CHUNKS = 2 * CHUNKS_PER_TILE - C0_CHUNKS
NBUF = 2
E_PAD = NW * EDGES_PER_TILE
N_CHUNKS = E_PAD // CHUNK
ROWS_PER_SUBCORE = N_PAD // NS
DEG_W = 1

_mesh = plsc.VectorSubcoreMesh(core_axis_name="c", subcore_axis_name="s")

_sc_params = pltpu.CompilerParams()
if "needs_layout_passes" in pltpu.CompilerParams.__dataclass_fields__:
    _sc_params = dataclasses.replace(_sc_params, needs_layout_passes=False)


@functools.partial(
    pl.kernel,
    mesh=_mesh,
    compiler_params=_sc_params,
    out_type=jax.ShapeDtypeStruct((NC * N_PAD,), jnp.float32),
    scratch_types=[
        pltpu.VMEM((CHUNK,), jnp.int32),
        pltpu.VMEM((N_PAD,), jnp.float32),
        pltpu.VMEM((NS * ROWS_PER_SUBCORE,), jnp.float32),
        pltpu.VMEM((ROWS_PER_SUBCORE,), jnp.float32),
        pltpu.VMEM_SHARED((NS * N_PAD,), jnp.float32),
    ],
)
def _deg_kernel(dst_hbm, deg_hbm, dst_v, cnt_v, gath_v, res_v, stage_sh):
    c = lax.axis_index("c")
    s = lax.axis_index("s")
    base = (c * NS + s) * EDGES_PER_TILE
    r0 = s * ROWS_PER_SUBCORE

    @pl.loop(0, N_PAD, step=16)
    def _(k):
        cnt_v.at[pl.ds(k, 16)][...] = jnp.zeros((16,), jnp.float32)

    ones = jnp.ones((16,), jnp.float32)

    @pl.loop(0, CHUNKS_PER_TILE)
    def _(j):
        pltpu.sync_copy(dst_hbm.at[pl.ds(base + j * CHUNK, CHUNK)], dst_v)

        @pl.loop(0, CHUNK, step=16)
        def _(k):
            plsc.addupdate_scatter(cnt_v, [dst_v[pl.ds(k, 16)]], ones)

    pltpu.sync_copy(cnt_v, stage_sh.at[pl.ds(s * N_PAD, N_PAD)])
    plsc.subcore_barrier()
    for r in range(NS):
        pltpu.sync_copy(
            stage_sh.at[pl.ds(r * N_PAD + r0, ROWS_PER_SUBCORE)],
            gath_v.at[pl.ds(r * ROWS_PER_SUBCORE, ROWS_PER_SUBCORE)])

    @pl.loop(0, ROWS_PER_SUBCORE, step=16)
    def _(k):
        acc = jnp.zeros((16,), jnp.float32)
        for r in range(NS):
            acc = acc + gath_v[pl.ds(r * ROWS_PER_SUBCORE + k, 16)]
        res_v.at[pl.ds(k, 16)][...] = acc

    pltpu.sync_copy(res_v, deg_hbm.at[pl.ds(c * N_PAD + r0, ROWS_PER_SUBCORE)])


@functools.partial(
    pl.kernel,
    mesh=_mesh,
    out_type=jax.ShapeDtypeStruct((NC * N_PAD, D), jnp.float32),
    scratch_types=(
        [pltpu.VMEM((CHUNK,), jnp.int32) for _ in range(2 * NBUF)]
        + [pltpu.VMEM((CHUNK, D), jnp.float32) for _ in range(NBUF)]
        + [pltpu.VMEM_SHARED((N_PAD, D), jnp.float32)]
        + [pltpu.SemaphoreType.DMA for _ in range(2 * NBUF)]
    ),
)
def _scatter_kernel(hs_hbm, src_hbm, dst_hbm, zero_hbm, out_hbm, *scratch):
    src_v = scratch[0:NBUF]
    dst_v = scratch[NBUF:2 * NBUF]
    rows_v = scratch[2 * NBUF:3 * NBUF]
    acc_sh = scratch[3 * NBUF]
    semg = scratch[3 * NBUF + 1:3 * NBUF + 1 + NBUF]
    semi = scratch[3 * NBUF + 1 + NBUF:3 * NBUF + 1 + 2 * NBUF]

    c = lax.axis_index("c")
    s = lax.axis_index("s")
    nch = jnp.where(c == 0, C0_CHUNKS, C1_CHUNKS)
    base = jnp.where(
        c == 0, s * (C0_CHUNKS * CHUNK),
        NS * (C0_CHUNKS * CHUNK) + s * (C1_CHUNKS * CHUNK))

    r0 = s * ROWS_PER_SUBCORE
    pltpu.sync_copy(zero_hbm, acc_sh.at[pl.ds(r0, ROWS_PER_SUBCORE)])
    plsc.subcore_barrier()

    def run(hs_hbm):
        for k in range(NBUF - 1):
            pltpu.sync_copy(src_hbm.at[pl.ds(base + k * CHUNK, CHUNK)], src_v[k])
            pltpu.sync_copy(dst_hbm.at[pl.ds(base + k * CHUNK, CHUNK)], dst_v[k])
            pltpu.async_copy(hs_hbm.at[src_v[k]], rows_v[k], semg[k])
        kb = NBUF - 1
        pltpu.async_copy(
            src_hbm.at[pl.ds(base + kb * CHUNK, CHUNK)], src_v[kb], semi[kb])
        pltpu.async_copy(
            dst_hbm.at[pl.ds(base + kb * CHUNK, CHUNK)], dst_v[kb], semi[kb])

        @pl.loop(0, nch, step=NBUF)
        def _(g):
            for b in range(NBUF):
                j = g + b
                o = (b + NBUF - 1) % NBUF

                @pl.when(j + NBUF - 1 < nch)
                def _():
                    off = base + (j + NBUF - 1) * CHUNK
                    pltpu.make_async_copy(
                        src_hbm.at[pl.ds(off, CHUNK)], src_v[o], semi[o]).wait()
                    pltpu.make_async_copy(
                        dst_hbm.at[pl.ds(off, CHUNK)], dst_v[o], semi[o]).wait()
                    pltpu.async_copy(hs_hbm.at[src_v[o]], rows_v[o], semg[o])

                pltpu.make_async_copy(
                    hs_hbm.at[src_v[b]], rows_v[b], semg[b]).wait()
                pltpu.sync_copy(rows_v[b], acc_sh.at[dst_v[b]], add=True)

                @pl.when(j + NBUF < nch)
                def _():
                    off = base + (j + NBUF) * CHUNK
                    pltpu.async_copy(
                        src_hbm.at[pl.ds(off, CHUNK)], src_v[b], semi[b])
                    pltpu.async_copy(
                        dst_hbm.at[pl.ds(off, CHUNK)], dst_v[b], semi[b])

    run(hs_hbm)

    plsc.subcore_barrier()
    pltpu.sync_copy(acc_sh.at[pl.ds(r0, ROWS_PER_SUBCORE)],
                    out_hbm.at[pl.ds(c * N_PAD + r0, ROWS_PER_SUBCORE)])


_BLK = 1280
_GRID = N_PAD // _BLK


def _row_spec(w=D):
    return pl.BlockSpec((_BLK, w), lambda i: (i, 0))


def _fixed_spec(shape):
    return pl.BlockSpec(shape, lambda i: tuple(0 for _ in shape))


def _dinv(dga_ref, dgb_ref):
    deg = dga_ref[...] + dgb_ref[...] + 1.0
    return lax.rsqrt(deg)


def _mm_body(x_ref, w_ref, o_ref):
    o_ref[...] = lax.dot_general(
        x_ref[...], w_ref[...], (((1,), (0,)), ((), ())),
        precision=lax.Precision.HIGHEST, preferred_element_type=jnp.float32)


_mm = pl.pallas_call(
    _mm_body,
    grid=(_GRID,),
    in_specs=[_row_spec(), _fixed_spec((D, D))],
    out_specs=_row_spec(),
    out_shape=jax.ShapeDtypeStruct((N_PAD, D), jnp.float32),
)


def _scale_body(h_ref, dga_ref, dgb_ref, o_ref):
    o_ref[...] = _dinv(dga_ref, dgb_ref) * h_ref[...]


_scale = pl.pallas_call(
    _scale_body,
    grid=(_GRID,),
    in_specs=[_row_spec(), _row_spec(DEG_W), _row_spec(DEG_W)],
    out_specs=_row_spec(),
    out_shape=jax.ShapeDtypeStruct((N_PAD, D), jnp.float32),
)


def _mid_body(sa_ref, sb_ref, hs_ref, dga_ref, dgb_ref, b_ref, w_ref, o_ref):
    dinv = _dinv(dga_ref, dgb_ref)
    t = jnp.maximum(dinv * (sa_ref[...] + sb_ref[...] + hs_ref[...]) + b_ref[...], 0.0)
    o_ref[...] = dinv * lax.dot_general(
        t, w_ref[...], (((1,), (0,)), ((), ())),
        precision=lax.Precision.HIGHEST, preferred_element_type=jnp.float32)


_mid = pl.pallas_call(
    _mid_body,
    grid=(_GRID,),
    in_specs=[_row_spec(), _row_spec(), _row_spec(), _row_spec(DEG_W),
              _row_spec(DEG_W), _fixed_spec((1, D)), _fixed_spec((D, D))],
    out_specs=_row_spec(),
    out_shape=jax.ShapeDtypeStruct((N_PAD, D), jnp.float32),
)


def _out_body(sa_ref, sb_ref, hs_ref, dga_ref, dgb_ref, b_ref, o_ref):
    dinv = _dinv(dga_ref, dgb_ref)
    o_ref[...] = dinv * (sa_ref[...] + sb_ref[...] + hs_ref[...]) + b_ref[...]


_out = pl.pallas_call(
    _out_body,
    grid=(_GRID,),
    in_specs=[_row_spec(), _row_spec(), _row_spec(), _row_spec(DEG_W),
              _row_spec(DEG_W), _fixed_spec((1, D))],
    out_specs=_row_spec(),
    out_shape=jax.ShapeDtypeStruct((N_PAD, D), jnp.float32),
)


def kernel(x, edge_index, W1, b1, W2, b2):
    src = edge_index[0].astype(jnp.int32)
    dst = edge_index[1].astype(jnp.int32)
    pad = jnp.full((E_PAD - E,), N, jnp.int32)
    src_p = jnp.concatenate([src, pad])
    dst_p = jnp.concatenate([dst, pad])
    x_p = jnp.pad(x, ((0, N_PAD - N), (0, 0)))
    zeroD = jnp.zeros((ROWS_PER_SUBCORE, D), jnp.float32)

    deg2 = _deg_kernel(dst_p).reshape(NC, N_PAD, DEG_W)
    dga, dgb = deg2[0], deg2[1]

    h1 = _mm(x_p, W1)
    hs1 = _scale(h1, dga, dgb)
    s1 = _scatter_kernel(hs1, src_p, dst_p, zeroD).reshape(NC, N_PAD, D)
    hs2 = _mid(s1[0], s1[1], hs1, dga, dgb, b1.reshape(1, D), W2)
    s2 = _scatter_kernel(hs2, src_p, dst_p, zeroD).reshape(NC, N_PAD, D)
    out = _out(s2[0], s2[1], hs2, dga, dgb, b2.reshape(1, D))
    return out[:N]


# speedup vs baseline: 1.2041x; 1.0143x over previous
import dataclasses
import functools

import jax
import jax.numpy as jnp
from jax import lax
from jax.experimental import pallas as pl
from jax.experimental.pallas import tpu as pltpu
from jax.experimental.pallas import tpu_sc as plsc

N = 10000
D = 128
N_PAD = 10240
E = 320000
NC, NS = 2, 16
NW = NC * NS
CHUNK = 128
EDGES_PER_TILE = 10240
CHUNKS_PER_TILE = EDGES_PER_TILE // CHUNK
C0_CHUNKS = 152
C1_CHUNKS = 2 * CHUNKS_PER_TILE - C0_CHUNKS
NBUF = 2
E_PAD = NW * EDGES_PER_TILE
N_CHUNKS = E_PAD // CHUNK
ROWS_PER_SUBCORE = N_PAD // NS
DEG_W = 1

_mesh = plsc.VectorSubcoreMesh(core_axis_name="c", subcore_axis_name="s")

_sc_params = pltpu.CompilerParams()
if "needs_layout_passes" in pltpu.CompilerParams.__dataclass_fields__:
    _sc_params = dataclasses.replace(_sc_params, needs_layout_passes=False)


@functools.partial(
    pl.kernel,
    mesh=_mesh,
    compiler_params=_sc_params,
    out_type=jax.ShapeDtypeStruct((NC * N_PAD,), jnp.float32),
    scratch_types=[
        pltpu.VMEM((CHUNK,), jnp.int32),
        pltpu.VMEM((N_PAD,), jnp.float32),
        pltpu.VMEM((NS * ROWS_PER_SUBCORE,), jnp.float32),
        pltpu.VMEM((ROWS_PER_SUBCORE,), jnp.float32),
        pltpu.VMEM_SHARED((NS * N_PAD,), jnp.float32),
    ],
)
def _deg_kernel(dst_hbm, deg_hbm, dst_v, cnt_v, gath_v, res_v, stage_sh):
    c = lax.axis_index("c")
    s = lax.axis_index("s")
    base = (c * NS + s) * EDGES_PER_TILE
    r0 = s * ROWS_PER_SUBCORE

    @pl.loop(0, N_PAD, step=16)
    def _(k):
        cnt_v.at[pl.ds(k, 16)][...] = jnp.zeros((16,), jnp.float32)

    ones = jnp.ones((16,), jnp.float32)

    @pl.loop(0, CHUNKS_PER_TILE)
    def _(j):
        pltpu.sync_copy(dst_hbm.at[pl.ds(base + j * CHUNK, CHUNK)], dst_v)

        @pl.loop(0, CHUNK, step=16)
        def _(k):
            plsc.addupdate_scatter(cnt_v, [dst_v[pl.ds(k, 16)]], ones)

    pltpu.sync_copy(cnt_v, stage_sh.at[pl.ds(s * N_PAD, N_PAD)])
    plsc.subcore_barrier()
    for r in range(NS):
        pltpu.sync_copy(
            stage_sh.at[pl.ds(r * N_PAD + r0, ROWS_PER_SUBCORE)],
            gath_v.at[pl.ds(r * ROWS_PER_SUBCORE, ROWS_PER_SUBCORE)])

    @pl.loop(0, ROWS_PER_SUBCORE, step=16)
    def _(k):
        acc = jnp.zeros((16,), jnp.float32)
        for r in range(NS):
            acc = acc + gath_v[pl.ds(r * ROWS_PER_SUBCORE + k, 16)]
        res_v.at[pl.ds(k, 16)][...] = acc

    pltpu.sync_copy(res_v, deg_hbm.at[pl.ds(c * N_PAD + r0, ROWS_PER_SUBCORE)])


@functools.partial(
    pl.kernel,
    mesh=_mesh,
    out_type=jax.ShapeDtypeStruct((NC * N_PAD, D), jnp.float32),
    scratch_types=(
        [pltpu.VMEM((CHUNK,), jnp.int32) for _ in range(2 * NBUF)]
        + [pltpu.VMEM((CHUNK, D), jnp.float32) for _ in range(NBUF)]
        + [pltpu.VMEM_SHARED((N_PAD, D), jnp.float32)]
        + [pltpu.SemaphoreType.DMA for _ in range(2 * NBUF)]
    ),
)
def _scatter_kernel(hs_hbm, src_hbm, dst_hbm, zero_hbm, out_hbm, *scratch):
    src_v = scratch[0:NBUF]
    dst_v = scratch[NBUF:2 * NBUF]
    rows_v = scratch[2 * NBUF:3 * NBUF]
    acc_sh = scratch[3 * NBUF]
    semg = scratch[3 * NBUF + 1:3 * NBUF + 1 + NBUF]
    semi = scratch[3 * NBUF + 1 + NBUF:3 * NBUF + 1 + 2 * NBUF]

    c = lax.axis_index("c")
    s = lax.axis_index("s")
    nch = jnp.where(c == 0, C0_CHUNKS, C1_CHUNKS)
    base = jnp.where(
        c == 0, s * (C0_CHUNKS * CHUNK),
        NS * (C0_CHUNKS * CHUNK) + s * (C1_CHUNKS * CHUNK))

    r0 = s * ROWS_PER_SUBCORE
    pltpu.sync_copy(zero_hbm, acc_sh.at[pl.ds(r0, ROWS_PER_SUBCORE)])
    plsc.subcore_barrier()

    def run(hs_hbm):
        for k in range(NBUF - 1):
            pltpu.sync_copy(src_hbm.at[pl.ds(base + k * CHUNK, CHUNK)], src_v[k])
            pltpu.sync_copy(dst_hbm.at[pl.ds(base + k * CHUNK, CHUNK)], dst_v[k])
            pltpu.async_copy(hs_hbm.at[src_v[k]], rows_v[k], semg[k])
        kb = NBUF - 1
        pltpu.async_copy(
            src_hbm.at[pl.ds(base + kb * CHUNK, CHUNK)], src_v[kb], semi[kb])
        pltpu.async_copy(
            dst_hbm.at[pl.ds(base + kb * CHUNK, CHUNK)], dst_v[kb], semi[kb])

        @pl.loop(0, nch, step=NBUF)
        def _(g):
            for b in range(NBUF):
                j = g + b
                o = (b + NBUF - 1) % NBUF

                @pl.when(j + NBUF - 1 < nch)
                def _():
                    off = base + (j + NBUF - 1) * CHUNK
                    pltpu.make_async_copy(
                        src_hbm.at[pl.ds(off, CHUNK)], src_v[o], semi[o]).wait()
                    pltpu.make_async_copy(
                        dst_hbm.at[pl.ds(off, CHUNK)], dst_v[o], semi[o]).wait()
                    pltpu.async_copy(hs_hbm.at[src_v[o]], rows_v[o], semg[o])

                pltpu.make_async_copy(
                    hs_hbm.at[src_v[b]], rows_v[b], semg[b]).wait()
                pltpu.sync_copy(rows_v[b], acc_sh.at[dst_v[b]], add=True)

                @pl.when(j + NBUF < nch)
                def _():
                    off = base + (j + NBUF) * CHUNK
                    pltpu.async_copy(
                        src_hbm.at[pl.ds(off, CHUNK)], src_v[b], semi[b])
                    pltpu.async_copy(
                        dst_hbm.at[pl.ds(off, CHUNK)], dst_v[b], semi[b])

    run(hs_hbm)

    plsc.subcore_barrier()
    pltpu.sync_copy(acc_sh.at[pl.ds(r0, ROWS_PER_SUBCORE)],
                    out_hbm.at[pl.ds(c * N_PAD + r0, ROWS_PER_SUBCORE)])


_BLK = 1280
_GRID = N_PAD // _BLK


def _row_spec(w=D):
    return pl.BlockSpec((_BLK, w), lambda i: (i, 0))


def _fixed_spec(shape):
    return pl.BlockSpec(shape, lambda i: tuple(0 for _ in shape))


def _dinv(dga_ref, dgb_ref):
    deg = dga_ref[...] + dgb_ref[...] + 1.0
    return lax.rsqrt(deg)


def _mm_body(x_ref, w_ref, o_ref):
    o_ref[...] = lax.dot_general(
        x_ref[...], w_ref[...], (((1,), (0,)), ((), ())),
        precision=lax.Precision.HIGHEST, preferred_element_type=jnp.float32)


_mm = pl.pallas_call(
    _mm_body,
    grid=(_GRID,),
    in_specs=[_row_spec(), _fixed_spec((D, D))],
    out_specs=_row_spec(),
    out_shape=jax.ShapeDtypeStruct((N_PAD, D), jnp.float32),
)


def _scale_body(h_ref, dga_ref, dgb_ref, o_ref):
    o_ref[...] = _dinv(dga_ref, dgb_ref) * h_ref[...]


_scale = pl.pallas_call(
    _scale_body,
    grid=(_GRID,),
    in_specs=[_row_spec(), _row_spec(DEG_W), _row_spec(DEG_W)],
    out_specs=_row_spec(),
    out_shape=jax.ShapeDtypeStruct((N_PAD, D), jnp.float32),
)


def _mid_body(sa_ref, sb_ref, hs_ref, dga_ref, dgb_ref, b_ref, w_ref, o_ref):
    dinv = _dinv(dga_ref, dgb_ref)
    t = jnp.maximum(dinv * (sa_ref[...] + sb_ref[...] + hs_ref[...]) + b_ref[...], 0.0)
    o_ref[...] = dinv * lax.dot_general(
        t, w_ref[...], (((1,), (0,)), ((), ())),
        precision=lax.Precision.HIGHEST, preferred_element_type=jnp.float32)


_mid = pl.pallas_call(
    _mid_body,
    grid=(_GRID,),
    in_specs=[_row_spec(), _row_spec(), _row_spec(), _row_spec(DEG_W),
              _row_spec(DEG_W), _fixed_spec((1, D)), _fixed_spec((D, D))],
    out_specs=_row_spec(),
    out_shape=jax.ShapeDtypeStruct((N_PAD, D), jnp.float32),
)


def _out_body(sa_ref, sb_ref, hs_ref, dga_ref, dgb_ref, b_ref, o_ref):
    dinv = _dinv(dga_ref, dgb_ref)
    o_ref[...] = dinv * (sa_ref[...] + sb_ref[...] + hs_ref[...]) + b_ref[...]


_out = pl.pallas_call(
    _out_body,
    grid=(_GRID,),
    in_specs=[_row_spec(), _row_spec(), _row_spec(), _row_spec(DEG_W),
              _row_spec(DEG_W), _fixed_spec((1, D))],
    out_specs=_row_spec(),
    out_shape=jax.ShapeDtypeStruct((N_PAD, D), jnp.float32),
)


def kernel(x, edge_index, W1, b1, W2, b2):
    src = edge_index[0].astype(jnp.int32)
    dst = edge_index[1].astype(jnp.int32)
    pad = jnp.full((E_PAD - E,), N, jnp.int32)
    src_p = jnp.concatenate([src, pad])
    dst_p = jnp.concatenate([dst, pad])
    x_p = jnp.pad(x, ((0, N_PAD - N), (0, 0)))
    zeroD = jnp.zeros((ROWS_PER_SUBCORE, D), jnp.float32)

    deg2 = _deg_kernel(dst_p).reshape(NC, N_PAD, DEG_W)
    dga, dgb = deg2[0], deg2[1]

    h1 = _mm(x_p, W1)
    hs1 = _scale(h1, dga, dgb)
    s1 = _scatter_kernel(hs1, src_p, dst_p, zeroD).reshape(NC, N_PAD, D)
    hs2 = _mid(s1[0], s1[1], hs1, dga, dgb, b1.reshape(1, D), W2)
    s2 = _scatter_kernel(hs2, src_p, dst_p, zeroD).reshape(NC, N_PAD, D)
    out = _out(s2[0], s2[1], hs2, dga, dgb, b2.reshape(1, D))
    return out[:N]
